# baseline (speedup 1.0000x reference)
import jax
import jax.numpy as jnp
from jax.experimental import pallas as pl
from jax.experimental.pallas import tpu as pltpu

G = 4
CE = 8
NM = 4


def _moe_body(x_hbm, gw_ref, wblk_ref, w_hbm, em_ref, ow_ref, ob_ref,
              gates_ref, y_ref, emo_ref,
              h_vmem, w_vmem, v_vmem, sems):
    k = pl.program_id(0)
    E = w_hbm.shape[0]
    lo = G * CE
    cm = (E - lo) // NM

    @pl.when(k == 0)
    def _start():
        pltpu.make_async_copy(
            x_hbm.at[:, 0, :], h_vmem, sems.at[NM]).start()
        for q in range(NM):
            pltpu.make_async_copy(
                w_hbm.at[pl.ds(lo + q * cm, cm)],
                w_vmem.at[pl.ds(q * cm, cm)], sems.at[q]).start()

    ow = ow_ref[...]

    v_vmem[pl.ds(k * CE, CE)] = jnp.sum(
        wblk_ref[...] * ow[None, :, :], axis=2)

    @pl.when(k == G - 1)
    def _finish():
        b = ob_ref[0, 0]

        vmin = jax.lax.dot_general(
            em_ref[...], ow, (((1,), (1,)), ((), ())),
            preferred_element_type=jnp.float32)

        pltpu.make_async_copy(
            x_hbm.at[:, 0, :], h_vmem, sems.at[NM]).wait()
        h = h_vmem[...]

        gates_ref[...] = jax.lax.dot_general(
            h, gw_ref[...], (((1,), (1,)), ((), ())),
            preferred_element_type=jnp.float32)
        emo_ref[...] = jax.lax.dot_general(
            h, vmin, (((1,), (0,)), ((), ()))) + b

        for q in range(NM):
            pltpu.make_async_copy(
                w_hbm.at[pl.ds(lo + q * cm, cm)],
                w_vmem.at[pl.ds(q * cm, cm)], sems.at[q]).wait()
            v_vmem[pl.ds(lo + q * cm, cm)] = jnp.sum(
                w_vmem[pl.ds(q * cm, cm)] * ow[None, :, :], axis=2)

        y_ref[...] = jax.lax.dot_general(
            h, v_vmem[...], (((1,), (1,)), ((), ())),
            preferred_element_type=jnp.float32) + b


def kernel(x, gate_weights, expert_weights, expert_min, out_w, out_b):
    B, _, H = x.shape
    E, _, S = expert_weights.shape
    ob2 = out_b.reshape(1, 1)

    gates, y2, emo = pl.pallas_call(
        _moe_body,
        grid=(G,),
        in_specs=[
            pl.BlockSpec(memory_space=pltpu.MemorySpace.HBM),
            pl.BlockSpec(memory_space=pltpu.VMEM),
            pl.BlockSpec((CE, H, S), lambda k: (k, 0, 0)),
            pl.BlockSpec(memory_space=pltpu.MemorySpace.HBM),
            pl.BlockSpec(memory_space=pltpu.VMEM),
            pl.BlockSpec(memory_space=pltpu.VMEM),
            pl.BlockSpec(memory_space=pltpu.VMEM),
        ],
        out_specs=[
            pl.BlockSpec((B, E), lambda k: (0, 0)),
            pl.BlockSpec((B, E), lambda k: (0, 0)),
            pl.BlockSpec((B, 1), lambda k: (0, 0)),
        ],
        out_shape=[
            jax.ShapeDtypeStruct((B, E), jnp.float32),
            jax.ShapeDtypeStruct((B, E), jnp.float32),
            jax.ShapeDtypeStruct((B, 1), jnp.float32),
        ],
        scratch_shapes=[
            pltpu.VMEM((B, H), jnp.float32),
            pltpu.VMEM((E - G * CE, H, S), jnp.float32),
            pltpu.VMEM((E, H), jnp.float32),
            pltpu.SemaphoreType.DMA((NM + 1,)),
        ],
    )(x, gate_weights, expert_weights, expert_weights, expert_min,
      out_w, ob2)

    return (gates, y2.reshape(B, E, 1), emo)
